# baseline (speedup 1.0000x reference)
import jax
import jax.numpy as jnp
from jax import lax
from jax.experimental import pallas as pl
from jax.experimental.pallas import tpu as pltpu
from jax.experimental.pallas import tpu_sc as plsc

N = 10000
E = 320000
D = 128
G = 256

NC = 2
NS = 16
NW = NC * NS
EW = E // NW
E_PAD = 327680
EWP = E_PAD // NW
CHUNK = 80
NCHUNK = EWP // CHUNK
N_PAD = 10240
RPS = N_PAD // NS
ZROWS = 128

RB = 400
NRB = N // RB


NB = 4


def _sc_scatter_body(x_hbm, src_hbm, dst_hbm, zeros_hbm, out_hbm,
                     srcb_v, dstb_v, rows_v, agg_sh, isem, dsem, gsem, ssem):
    cid = lax.axis_index("c")
    sid = lax.axis_index("s")
    wid = sid * NC + cid

    for j in range(RPS // ZROWS):
        pltpu.sync_copy(zeros_hbm,
                        agg_sh.at[pl.ds(sid * RPS + j * ZROWS, ZROWS)])
    plsc.subcore_barrier()

    def round_(j, carry):
        base = wid * EWP + j * NB * CHUNK
        for b in range(NB):
            pltpu.async_copy(src_hbm.at[pl.ds(base + b * CHUNK, CHUNK)],
                             srcb_v.at[b], isem)
            pltpu.async_copy(dst_hbm.at[pl.ds(base + b * CHUNK, CHUNK)],
                             dstb_v.at[b], dsem)
        for b in range(NB):
            pltpu.make_async_copy(src_hbm.at[pl.ds(base + b * CHUNK, CHUNK)],
                                  srcb_v.at[b], isem).wait()
            pltpu.async_copy(x_hbm.at[srcb_v.at[b]], rows_v.at[b], gsem)
        for b in range(NB):
            pltpu.make_async_copy(x_hbm.at[srcb_v.at[b]],
                                  rows_v.at[b], gsem).wait()
            pltpu.make_async_copy(dst_hbm.at[pl.ds(base + b * CHUNK, CHUNK)],
                                  dstb_v.at[b], dsem).wait()
            pltpu.async_copy(rows_v.at[b], agg_sh.at[dstb_v.at[b]],
                             ssem, add=True)
        for b in range(NB):
            pltpu.make_async_copy(rows_v.at[b], agg_sh.at[dstb_v.at[b]],
                                  ssem).wait()
        return carry

    lax.fori_loop(0, NCHUNK // NB, round_, 0)
    plsc.subcore_barrier()

    pltpu.sync_copy(agg_sh.at[pl.ds(sid * RPS, RPS)],
                    out_hbm.at[cid, pl.ds(sid * RPS, RPS)])


import functools


@functools.cache
def _get_sc_scatter():
    return pl.kernel(
        _sc_scatter_body,
        out_type=jax.ShapeDtypeStruct((NC, N_PAD, D), jnp.float32),
        mesh=plsc.VectorSubcoreMesh(core_axis_name="c", subcore_axis_name="s",
                                    num_cores=NC, num_subcores=NS),
        scratch_types=[
            pltpu.VMEM((NB, CHUNK), jnp.int32),
            pltpu.VMEM((NB, CHUNK), jnp.int32),
            pltpu.VMEM((NB, CHUNK, D), jnp.float32),
            pltpu.VMEM_SHARED((N_PAD, D), jnp.float32),
            pltpu.SemaphoreType.DMA,
            pltpu.SemaphoreType.DMA,
            pltpu.SemaphoreType.DMA,
            pltpu.SemaphoreType.DMA,
        ],
    )


def _sc_scatter(x, src, dst, zeros):
    return _get_sc_scatter()(x, src, dst, zeros)


def _mlp_body(x_ref, p0_ref, p1_ref, w1_ref, b1_ref, w2_ref, b2_ref, o_ref):
    h = x_ref[...] + p0_ref[0] + p1_ref[0]
    h = jnp.maximum(
        jnp.dot(h, w1_ref[...], preferred_element_type=jnp.float32)
        + b1_ref[...], 0.0)
    h = jnp.dot(h, w2_ref[...], preferred_element_type=jnp.float32) + b2_ref[...]
    o_ref[...] = jnp.maximum(h, 0.0)


def _mlp(xp, P, w1, b1, w2, b2):
    row = pl.BlockSpec((RB, D), lambda i: (i, 0))
    p0s = pl.BlockSpec((1, RB, D), lambda i: (0, i, 0))
    p1s = pl.BlockSpec((1, RB, D), lambda i: (1, i, 0))
    full = pl.BlockSpec((D, D), lambda i: (0, 0))
    vec = pl.BlockSpec((1, D), lambda i: (0, 0))
    return pl.pallas_call(
        _mlp_body,
        grid=(NRB,),
        in_specs=[row, p0s, p1s, full, vec, full, vec],
        out_specs=row,
        out_shape=jax.ShapeDtypeStruct((N, D), jnp.float32),
    )(xp, P, P, w1, b1, w2, b2)


def _final_body(h_ref, q0_ref, q1_ref, b_ref, w1_ref, b1_ref, w2_ref, b2_ref,
                fcw_ref, fcb_ref, hw_ref, hb_ref, o_ref, acc_ref, cnt_ref):
    i = pl.program_id(0)

    @pl.when(i == 0)
    def _():
        acc_ref[...] = jnp.zeros_like(acc_ref)
        cnt_ref[...] = jnp.zeros_like(cnt_ref)

    h = h_ref[...] + q0_ref[0] + q1_ref[0]
    h = jnp.maximum(
        jnp.dot(h, w1_ref[...], preferred_element_type=jnp.float32)
        + b1_ref[...], 0.0)
    h = jnp.dot(h, w2_ref[...], preferred_element_type=jnp.float32) + b2_ref[...]
    h = jnp.maximum(h, 0.0)

    seg = b_ref[0]
    gids = lax.broadcasted_iota(jnp.int32, (G, RB), 0)
    onehot = (gids == seg).astype(jnp.float32)
    acc_ref[...] += jnp.dot(onehot, h, preferred_element_type=jnp.float32)
    cnt_ref[...] += jnp.sum(onehot, axis=1, keepdims=True)

    @pl.when(i == NRB - 1)
    def _():
        counts = jnp.maximum(cnt_ref[...][:, 0:1], 1.0)
        pooled = acc_ref[...] / counts
        s = jnp.maximum(
            jnp.dot(pooled, fcw_ref[...], preferred_element_type=jnp.float32)
            + fcb_ref[...], 0.0)
        o_ref[...] = (jnp.dot(s, hw_ref[...], preferred_element_type=jnp.float32)
                      + hb_ref[...])


def _final(h1, Q, batch3, w1, b1, w2, b2, fcw, fcb, hw, hb):
    row = pl.BlockSpec((RB, D), lambda i: (i, 0))
    q0s = pl.BlockSpec((1, RB, D), lambda i: (0, i, 0))
    q1s = pl.BlockSpec((1, RB, D), lambda i: (1, i, 0))
    full = pl.BlockSpec((D, D), lambda i: (0, 0))
    vec = pl.BlockSpec((1, D), lambda i: (0, 0))
    return pl.pallas_call(
        _final_body,
        grid=(NRB,),
        in_specs=[
            row, q0s, q1s,
            pl.BlockSpec((1, 1, RB), lambda i: (i, 0, 0)),
            full, vec, full, vec,
            full, vec, full, vec,
        ],
        out_specs=pl.BlockSpec((G, D), lambda i: (0, 0)),
        out_shape=jax.ShapeDtypeStruct((G, D), jnp.float32),
        scratch_shapes=[
            pltpu.VMEM((G, D), jnp.float32),
            pltpu.VMEM((G, D), jnp.float32),
        ],
    )(h1, Q, Q, batch3, w1, b1, w2, b2, fcw, fcb, hw, hb)


def kernel(x, edge_index, batch,
           c1_w1, c1_b1, c1_w2, c1_b2,
           c2_w1, c2_b1, c2_w2, c2_b2,
           fc_w, fc_b, hS_w, hS_b, hP_w, hP_b, hN_w, hN_b):
    src = jnp.concatenate(
        [edge_index[0], jnp.zeros(E_PAD - E, jnp.int32)])
    dst = jnp.concatenate(
        [edge_index[1], jnp.full(E_PAD - E, N_PAD - 1, jnp.int32)])
    zeros = jnp.zeros((ZROWS, D), jnp.float32)

    P = _sc_scatter(x, src, dst, zeros)
    h1 = _mlp(x, P,
              c1_w1, c1_b1.reshape(1, D), c1_w2, c1_b2.reshape(1, D))
    Q = _sc_scatter(h1, src, dst, zeros)

    batch3 = batch.reshape(NRB, 1, RB)
    fcw = jnp.pad(fc_w, ((0, 0), (0, D - fc_w.shape[1])))
    fcb = jnp.pad(fc_b, (0, D - fc_b.shape[0])).reshape(1, D)
    hw = jnp.concatenate([hS_w, hP_w, hN_w], axis=1)
    hw = jnp.pad(hw, ((0, D - hw.shape[0]), (0, D - hw.shape[1])))
    hb = jnp.concatenate([hS_b, hP_b, hN_b], axis=0)
    hb = jnp.pad(hb, (0, D - hb.shape[0])).reshape(1, D)

    o = _final(h1, Q, batch3,
               c2_w1, c2_b1.reshape(1, D), c2_w2, c2_b2.reshape(1, D),
               fcw, fcb, hw, hb)
    return o[:, 0], o[:, 1], o[:, 2]


# speedup vs baseline: 1.0004x; 1.0004x over previous
import jax
import jax.numpy as jnp
from jax import lax
from jax.experimental import pallas as pl
from jax.experimental.pallas import tpu as pltpu
from jax.experimental.pallas import tpu_sc as plsc

N = 10000
E = 320000
D = 128
G = 256

NC = 2
NS = 16
NW = NC * NS
EW = E // NW
E_PAD = 327680
EWP = E_PAD // NW
CHUNK = 80
NCHUNK = EWP // CHUNK
N_PAD = 10240
RPS = N_PAD // NS
ZROWS = 128

RB = 400
NRB = N // RB


NB = 4


def _sc_scatter_body(x_hbm, src_hbm, dst_hbm, zeros_hbm, out_hbm,
                     s0, s1, s2, s3, d0, d1, d2, d3, r0, r1, r2, r3,
                     agg_sh, isem, dsem, gsem, ssem):
    cid = lax.axis_index("c")
    sid = lax.axis_index("s")
    wid = sid * NC + cid
    srcb = [s0, s1, s2, s3]
    dstb = [d0, d1, d2, d3]
    rows = [r0, r1, r2, r3]

    for j in range(RPS // ZROWS):
        pltpu.sync_copy(zeros_hbm,
                        agg_sh.at[pl.ds(sid * RPS + j * ZROWS, ZROWS)])
    plsc.subcore_barrier()

    def round_(j, carry):
        base = wid * EWP + j * (NB * CHUNK)
        for b in range(NB):
            pltpu.async_copy(src_hbm.at[pl.ds(base + b * CHUNK, CHUNK)],
                             srcb[b], isem)
            pltpu.async_copy(dst_hbm.at[pl.ds(base + b * CHUNK, CHUNK)],
                             dstb[b], dsem)
        for b in range(NB):
            pltpu.make_async_copy(src_hbm.at[pl.ds(base + b * CHUNK, CHUNK)],
                                  srcb[b], isem).wait()
            pltpu.async_copy(x_hbm.at[srcb[b]], rows[b], gsem)
        for b in range(NB):
            pltpu.make_async_copy(x_hbm.at[srcb[b]], rows[b], gsem).wait()
            pltpu.make_async_copy(dst_hbm.at[pl.ds(base + b * CHUNK, CHUNK)],
                                  dstb[b], dsem).wait()
            pltpu.async_copy(rows[b], agg_sh.at[dstb[b]], ssem, add=True)
        for b in range(NB):
            pltpu.make_async_copy(rows[b], agg_sh.at[dstb[b]], ssem).wait()
        return carry

    lax.fori_loop(0, NCHUNK // NB, round_, 0)
    plsc.subcore_barrier()

    pltpu.sync_copy(agg_sh.at[pl.ds(sid * RPS, RPS)],
                    out_hbm.at[cid, pl.ds(sid * RPS, RPS)])


import functools


@functools.cache
def _get_sc_scatter():
    return pl.kernel(
        _sc_scatter_body,
        out_type=jax.ShapeDtypeStruct((NC, N_PAD, D), jnp.float32),
        mesh=plsc.VectorSubcoreMesh(core_axis_name="c", subcore_axis_name="s",
                                    num_cores=NC, num_subcores=NS),
        scratch_types=(
            [pltpu.VMEM((CHUNK,), jnp.int32) for _ in range(2 * NB)]
            + [pltpu.VMEM((CHUNK, D), jnp.float32) for _ in range(NB)]
            + [pltpu.VMEM_SHARED((N_PAD, D), jnp.float32),
               pltpu.SemaphoreType.DMA,
               pltpu.SemaphoreType.DMA,
               pltpu.SemaphoreType.DMA,
               pltpu.SemaphoreType.DMA]),
    )


def _sc_scatter(x, src, dst, zeros):
    return _get_sc_scatter()(x, src, dst, zeros)


def _mlp_body(x_ref, p0_ref, p1_ref, w1_ref, b1_ref, w2_ref, b2_ref, o_ref):
    h = x_ref[...] + p0_ref[0] + p1_ref[0]
    h = jnp.maximum(
        jnp.dot(h, w1_ref[...], preferred_element_type=jnp.float32)
        + b1_ref[...], 0.0)
    h = jnp.dot(h, w2_ref[...], preferred_element_type=jnp.float32) + b2_ref[...]
    o_ref[...] = jnp.maximum(h, 0.0)


def _mlp(xp, P, w1, b1, w2, b2):
    row = pl.BlockSpec((RB, D), lambda i: (i, 0))
    p0s = pl.BlockSpec((1, RB, D), lambda i: (0, i, 0))
    p1s = pl.BlockSpec((1, RB, D), lambda i: (1, i, 0))
    full = pl.BlockSpec((D, D), lambda i: (0, 0))
    vec = pl.BlockSpec((1, D), lambda i: (0, 0))
    return pl.pallas_call(
        _mlp_body,
        grid=(NRB,),
        in_specs=[row, p0s, p1s, full, vec, full, vec],
        out_specs=row,
        out_shape=jax.ShapeDtypeStruct((N, D), jnp.float32),
    )(xp, P, P, w1, b1, w2, b2)


def _final_body(h_ref, q0_ref, q1_ref, b_ref, w1_ref, b1_ref, w2_ref, b2_ref,
                fcw_ref, fcb_ref, hw_ref, hb_ref, o_ref, acc_ref, cnt_ref):
    i = pl.program_id(0)

    @pl.when(i == 0)
    def _():
        acc_ref[...] = jnp.zeros_like(acc_ref)
        cnt_ref[...] = jnp.zeros_like(cnt_ref)

    h = h_ref[...] + q0_ref[0] + q1_ref[0]
    h = jnp.maximum(
        jnp.dot(h, w1_ref[...], preferred_element_type=jnp.float32)
        + b1_ref[...], 0.0)
    h = jnp.dot(h, w2_ref[...], preferred_element_type=jnp.float32) + b2_ref[...]
    h = jnp.maximum(h, 0.0)

    seg = b_ref[0]
    gids = lax.broadcasted_iota(jnp.int32, (G, RB), 0)
    onehot = (gids == seg).astype(jnp.float32)
    acc_ref[...] += jnp.dot(onehot, h, preferred_element_type=jnp.float32)
    cnt_ref[...] += jnp.sum(onehot, axis=1, keepdims=True)

    @pl.when(i == NRB - 1)
    def _():
        counts = jnp.maximum(cnt_ref[...][:, 0:1], 1.0)
        pooled = acc_ref[...] / counts
        s = jnp.maximum(
            jnp.dot(pooled, fcw_ref[...], preferred_element_type=jnp.float32)
            + fcb_ref[...], 0.0)
        o_ref[...] = (jnp.dot(s, hw_ref[...], preferred_element_type=jnp.float32)
                      + hb_ref[...])


def _final(h1, Q, batch3, w1, b1, w2, b2, fcw, fcb, hw, hb):
    row = pl.BlockSpec((RB, D), lambda i: (i, 0))
    q0s = pl.BlockSpec((1, RB, D), lambda i: (0, i, 0))
    q1s = pl.BlockSpec((1, RB, D), lambda i: (1, i, 0))
    full = pl.BlockSpec((D, D), lambda i: (0, 0))
    vec = pl.BlockSpec((1, D), lambda i: (0, 0))
    return pl.pallas_call(
        _final_body,
        grid=(NRB,),
        in_specs=[
            row, q0s, q1s,
            pl.BlockSpec((1, 1, RB), lambda i: (i, 0, 0)),
            full, vec, full, vec,
            full, vec, full, vec,
        ],
        out_specs=pl.BlockSpec((G, D), lambda i: (0, 0)),
        out_shape=jax.ShapeDtypeStruct((G, D), jnp.float32),
        scratch_shapes=[
            pltpu.VMEM((G, D), jnp.float32),
            pltpu.VMEM((G, D), jnp.float32),
        ],
    )(h1, Q, Q, batch3, w1, b1, w2, b2, fcw, fcb, hw, hb)


def kernel(x, edge_index, batch,
           c1_w1, c1_b1, c1_w2, c1_b2,
           c2_w1, c2_b1, c2_w2, c2_b2,
           fc_w, fc_b, hS_w, hS_b, hP_w, hP_b, hN_w, hN_b):
    src = jnp.concatenate(
        [edge_index[0], jnp.zeros(E_PAD - E, jnp.int32)])
    dst = jnp.concatenate(
        [edge_index[1], jnp.full(E_PAD - E, N_PAD - 1, jnp.int32)])
    zeros = jnp.zeros((ZROWS, D), jnp.float32)

    P = _sc_scatter(x, src, dst, zeros)
    h1 = _mlp(x, P,
              c1_w1, c1_b1.reshape(1, D), c1_w2, c1_b2.reshape(1, D))
    Q = _sc_scatter(h1, src, dst, zeros)

    batch3 = batch.reshape(NRB, 1, RB)
    fcw = jnp.pad(fc_w, ((0, 0), (0, D - fc_w.shape[1])))
    fcb = jnp.pad(fc_b, (0, D - fc_b.shape[0])).reshape(1, D)
    hw = jnp.concatenate([hS_w, hP_w, hN_w], axis=1)
    hw = jnp.pad(hw, ((0, D - hw.shape[0]), (0, D - hw.shape[1])))
    hb = jnp.concatenate([hS_b, hP_b, hN_b], axis=0)
    hb = jnp.pad(hb, (0, D - hb.shape[0])).reshape(1, D)

    o = _final(h1, Q, batch3,
               c2_w1, c2_b1.reshape(1, D), c2_w2, c2_b2.reshape(1, D),
               fcw, fcb, hw, hb)
    return o[:, 0], o[:, 1], o[:, 2]


# speedup vs baseline: 2.7449x; 2.7437x over previous
import jax
import jax.numpy as jnp
from jax import lax
from jax.experimental import pallas as pl
from jax.experimental.pallas import tpu as pltpu
from jax.experimental.pallas import tpu_sc as plsc

N = 10000
E = 320000
D = 128
G = 256

NC = 2
NS = 16
NW = NC * NS
EW = E // NW
E_PAD = 327680
EWP = E_PAD // NW
CHUNK = 80
NCHUNK = EWP // CHUNK
N_PAD = 10240
RPS = N_PAD // NS
ZROWS = 128

RB = 400
NRB = N // RB


NB = 4


def _sc_scatter_body(x_hbm, src_hbm, dst_hbm, zeros_hbm, out_hbm,
                     s0, s1, s2, s3, d0, d1, d2, d3, r0, r1, r2, r3,
                     agg_sh, isems, dsems, gsems, ssems):
    cid = lax.axis_index("c")
    sid = lax.axis_index("s")
    wid = sid * NC + cid
    srcb = [s0, s1, s2, s3]
    dstb = [d0, d1, d2, d3]
    rows = [r0, r1, r2, r3]

    for j in range(RPS // ZROWS):
        pltpu.sync_copy(zeros_hbm,
                        agg_sh.at[pl.ds(sid * RPS + j * ZROWS, ZROWS)])
    plsc.subcore_barrier()

    def round_(j, carry):
        base = wid * EWP + j * (NB * CHUNK)
        for b in range(NB):
            pltpu.async_copy(src_hbm.at[pl.ds(base + b * CHUNK, CHUNK)],
                             srcb[b], isems.at[b])
            pltpu.async_copy(dst_hbm.at[pl.ds(base + b * CHUNK, CHUNK)],
                             dstb[b], dsems.at[b])
        for b in range(NB):
            pltpu.make_async_copy(src_hbm.at[pl.ds(base + b * CHUNK, CHUNK)],
                                  srcb[b], isems.at[b]).wait()
            pltpu.async_copy(x_hbm.at[srcb[b]], rows[b], gsems.at[b])
        for b in range(NB):
            pltpu.make_async_copy(x_hbm.at[srcb[b]], rows[b],
                                  gsems.at[b]).wait()
            pltpu.make_async_copy(dst_hbm.at[pl.ds(base + b * CHUNK, CHUNK)],
                                  dstb[b], dsems.at[b]).wait()
            pltpu.async_copy(rows[b], agg_sh.at[dstb[b]], ssems.at[b],
                             add=True)
        for b in range(NB):
            pltpu.make_async_copy(rows[b], agg_sh.at[dstb[b]],
                                  ssems.at[b]).wait()
        return carry

    lax.fori_loop(0, NCHUNK // NB, round_, 0)
    plsc.subcore_barrier()

    pltpu.sync_copy(agg_sh.at[pl.ds(sid * RPS, RPS)],
                    out_hbm.at[cid, pl.ds(sid * RPS, RPS)])


import functools


@functools.cache
def _get_sc_scatter():
    return pl.kernel(
        _sc_scatter_body,
        out_type=jax.ShapeDtypeStruct((NC, N_PAD, D), jnp.float32),
        mesh=plsc.VectorSubcoreMesh(core_axis_name="c", subcore_axis_name="s",
                                    num_cores=NC, num_subcores=NS),
        scratch_types=(
            [pltpu.VMEM((CHUNK,), jnp.int32) for _ in range(2 * NB)]
            + [pltpu.VMEM((CHUNK, D), jnp.float32) for _ in range(NB)]
            + [pltpu.VMEM_SHARED((N_PAD, D), jnp.float32),
               pltpu.SemaphoreType.DMA((NB,)),
               pltpu.SemaphoreType.DMA((NB,)),
               pltpu.SemaphoreType.DMA((NB,)),
               pltpu.SemaphoreType.DMA((NB,))]),
    )


def _sc_scatter(x, src, dst, zeros):
    return _get_sc_scatter()(x, src, dst, zeros)


def _mlp_body(x_ref, p0_ref, p1_ref, w1_ref, b1_ref, w2_ref, b2_ref, o_ref):
    h = x_ref[...] + p0_ref[0] + p1_ref[0]
    h = jnp.maximum(
        jnp.dot(h, w1_ref[...], preferred_element_type=jnp.float32)
        + b1_ref[...], 0.0)
    h = jnp.dot(h, w2_ref[...], preferred_element_type=jnp.float32) + b2_ref[...]
    o_ref[...] = jnp.maximum(h, 0.0)


def _mlp(xp, P, w1, b1, w2, b2):
    row = pl.BlockSpec((RB, D), lambda i: (i, 0))
    p0s = pl.BlockSpec((1, RB, D), lambda i: (0, i, 0))
    p1s = pl.BlockSpec((1, RB, D), lambda i: (1, i, 0))
    full = pl.BlockSpec((D, D), lambda i: (0, 0))
    vec = pl.BlockSpec((1, D), lambda i: (0, 0))
    return pl.pallas_call(
        _mlp_body,
        grid=(NRB,),
        in_specs=[row, p0s, p1s, full, vec, full, vec],
        out_specs=row,
        out_shape=jax.ShapeDtypeStruct((N, D), jnp.float32),
    )(xp, P, P, w1, b1, w2, b2)


def _final_body(h_ref, q0_ref, q1_ref, b_ref, w1_ref, b1_ref, w2_ref, b2_ref,
                fcw_ref, fcb_ref, hw_ref, hb_ref, o_ref, acc_ref, cnt_ref):
    i = pl.program_id(0)

    @pl.when(i == 0)
    def _():
        acc_ref[...] = jnp.zeros_like(acc_ref)
        cnt_ref[...] = jnp.zeros_like(cnt_ref)

    h = h_ref[...] + q0_ref[0] + q1_ref[0]
    h = jnp.maximum(
        jnp.dot(h, w1_ref[...], preferred_element_type=jnp.float32)
        + b1_ref[...], 0.0)
    h = jnp.dot(h, w2_ref[...], preferred_element_type=jnp.float32) + b2_ref[...]
    h = jnp.maximum(h, 0.0)

    seg = b_ref[0]
    gids = lax.broadcasted_iota(jnp.int32, (G, RB), 0)
    onehot = (gids == seg).astype(jnp.float32)
    acc_ref[...] += jnp.dot(onehot, h, preferred_element_type=jnp.float32)
    cnt_ref[...] += jnp.sum(onehot, axis=1, keepdims=True)

    @pl.when(i == NRB - 1)
    def _():
        counts = jnp.maximum(cnt_ref[...][:, 0:1], 1.0)
        pooled = acc_ref[...] / counts
        s = jnp.maximum(
            jnp.dot(pooled, fcw_ref[...], preferred_element_type=jnp.float32)
            + fcb_ref[...], 0.0)
        o_ref[...] = (jnp.dot(s, hw_ref[...], preferred_element_type=jnp.float32)
                      + hb_ref[...])


def _final(h1, Q, batch3, w1, b1, w2, b2, fcw, fcb, hw, hb):
    row = pl.BlockSpec((RB, D), lambda i: (i, 0))
    q0s = pl.BlockSpec((1, RB, D), lambda i: (0, i, 0))
    q1s = pl.BlockSpec((1, RB, D), lambda i: (1, i, 0))
    full = pl.BlockSpec((D, D), lambda i: (0, 0))
    vec = pl.BlockSpec((1, D), lambda i: (0, 0))
    return pl.pallas_call(
        _final_body,
        grid=(NRB,),
        in_specs=[
            row, q0s, q1s,
            pl.BlockSpec((1, 1, RB), lambda i: (i, 0, 0)),
            full, vec, full, vec,
            full, vec, full, vec,
        ],
        out_specs=pl.BlockSpec((G, D), lambda i: (0, 0)),
        out_shape=jax.ShapeDtypeStruct((G, D), jnp.float32),
        scratch_shapes=[
            pltpu.VMEM((G, D), jnp.float32),
            pltpu.VMEM((G, D), jnp.float32),
        ],
    )(h1, Q, Q, batch3, w1, b1, w2, b2, fcw, fcb, hw, hb)


def kernel(x, edge_index, batch,
           c1_w1, c1_b1, c1_w2, c1_b2,
           c2_w1, c2_b1, c2_w2, c2_b2,
           fc_w, fc_b, hS_w, hS_b, hP_w, hP_b, hN_w, hN_b):
    pad = jnp.arange(E_PAD - E, dtype=jnp.int32)
    src = jnp.concatenate([edge_index[0], pad % N])
    dst = jnp.concatenate([edge_index[1], N + pad % (N_PAD - N)])
    zeros = jnp.zeros((ZROWS, D), jnp.float32)

    P = _sc_scatter(x, src, dst, zeros)
    h1 = _mlp(x, P,
              c1_w1, c1_b1.reshape(1, D), c1_w2, c1_b2.reshape(1, D))
    Q = _sc_scatter(h1, src, dst, zeros)

    batch3 = batch.reshape(NRB, 1, RB)
    fcw = jnp.pad(fc_w, ((0, 0), (0, D - fc_w.shape[1])))
    fcb = jnp.pad(fc_b, (0, D - fc_b.shape[0])).reshape(1, D)
    hw = jnp.concatenate([hS_w, hP_w, hN_w], axis=1)
    hw = jnp.pad(hw, ((0, D - hw.shape[0]), (0, D - hw.shape[1])))
    hb = jnp.concatenate([hS_b, hP_b, hN_b], axis=0)
    hb = jnp.pad(hb, (0, D - hb.shape[0])).reshape(1, D)

    o = _final(h1, Q, batch3,
               c2_w1, c2_b1.reshape(1, D), c2_w2, c2_b2.reshape(1, D),
               fcw, fcb, hw, hb)
    return o[:, 0], o[:, 1], o[:, 2]


# speedup vs baseline: 3.1954x; 1.1641x over previous
import jax
import jax.numpy as jnp
from jax import lax
from jax.experimental import pallas as pl
from jax.experimental.pallas import tpu as pltpu
from jax.experimental.pallas import tpu_sc as plsc

N = 10000
E = 320000
D = 128
G = 256

NC = 2
NS = 16
NW = NC * NS
EW = E // NW
E_PAD = 327680
EWP = E_PAD // NW
CHUNK = 80
NCHUNK = EWP // CHUNK
N_PAD = 10240
RPS = N_PAD // NS
ZROWS = 128

RB = 400
NRB = N // RB


NB = 4


NROUND = NCHUNK // NB
RIDX = NB * CHUNK


def _sc_scatter_body(x_hbm, src_hbm, dst_hbm, zeros_hbm, out_hbm,
                     srcB, dstB, r0, r1, r2, r3,
                     agg_sh, isems, dsems, gsems, ssems):
    cid = lax.axis_index("c")
    sid = lax.axis_index("s")
    wid = sid * NC + cid
    rows = [r0, r1, r2, r3]

    for j in range(RPS // ZROWS):
        pltpu.sync_copy(zeros_hbm,
                        agg_sh.at[pl.ds(sid * RPS + j * ZROWS, ZROWS)])

    def fetch_src(p, r):
        base = wid * EWP + r * RIDX
        pltpu.async_copy(src_hbm.at[pl.ds(base, RIDX)],
                         srcB.at[pl.ds(p * RIDX, RIDX)], isems.at[p])

    def wait_src(p, r):
        base = wid * EWP + r * RIDX
        pltpu.make_async_copy(src_hbm.at[pl.ds(base, RIDX)],
                              srcB.at[pl.ds(p * RIDX, RIDX)],
                              isems.at[p]).wait()

    def fetch_dst(p, r, b):
        base = wid * EWP + r * RIDX + b * CHUNK
        pltpu.async_copy(dst_hbm.at[pl.ds(base, CHUNK)],
                         dstB.at[p * NB + b], dsems.at[p])

    def wait_dst(p, r, b):
        base = wid * EWP + r * RIDX + b * CHUNK
        pltpu.make_async_copy(dst_hbm.at[pl.ds(base, CHUNK)],
                              dstB.at[p * NB + b], dsems.at[p]).wait()

    def start_gather(p, b):
        pltpu.async_copy(
            x_hbm.at[srcB.at[pl.ds(p * RIDX + b * CHUNK, CHUNK)]],
            rows[b], gsems.at[b])

    def wait_gather(p, b):
        pltpu.make_async_copy(
            x_hbm.at[srcB.at[pl.ds(p * RIDX + b * CHUNK, CHUNK)]],
            rows[b], gsems.at[b]).wait()

    def start_scatter(p, b):
        pltpu.async_copy(rows[b], agg_sh.at[dstB.at[p * NB + b]],
                         ssems.at[b], add=True)

    def wait_scatter(p, b):
        pltpu.make_async_copy(rows[b], agg_sh.at[dstB.at[p * NB + b]],
                              ssems.at[b]).wait()

    fetch_src(0, 0)
    fetch_src(1, 1)
    for b in range(NB):
        fetch_dst(0, 0, b)
        fetch_dst(1, 1, b)
    plsc.subcore_barrier()
    wait_src(0, 0)
    for b in range(NB):
        start_gather(0, b)

    def half(p, r, rn):
        for b in range(NB):
            wait_gather(p, b)
        for b in range(NB):
            wait_dst(p, r, b)
        for b in range(NB):
            start_scatter(p, b)
        fetch_src(p, rn)
        wait_src(1 - p, r + 1)
        for b in range(NB):
            wait_scatter(p, b)
            start_gather(1 - p, b)
        for b in range(NB):
            fetch_dst(p, rn, b)

    def body(t, carry):
        half(0, 2 * t, 2 * t + 2)
        half(1, 2 * t + 1, 2 * t + 3)
        return carry

    lax.fori_loop(0, NROUND // 2 - 1, body, 0)

    rl0, rl1 = NROUND - 2, NROUND - 1
    for b in range(NB):
        wait_gather(0, b)
    for b in range(NB):
        wait_dst(0, rl0, b)
    for b in range(NB):
        start_scatter(0, b)
    wait_src(1, rl1)
    for b in range(NB):
        wait_scatter(0, b)
        start_gather(1, b)
    for b in range(NB):
        wait_gather(1, b)
    for b in range(NB):
        wait_dst(1, rl1, b)
    for b in range(NB):
        start_scatter(1, b)
    for b in range(NB):
        wait_scatter(1, b)

    plsc.subcore_barrier()
    pltpu.sync_copy(agg_sh.at[pl.ds(sid * RPS, RPS)],
                    out_hbm.at[cid, pl.ds(sid * RPS, RPS)])


import functools


@functools.cache
def _get_sc_scatter():
    return pl.kernel(
        _sc_scatter_body,
        out_type=jax.ShapeDtypeStruct((NC, N_PAD, D), jnp.float32),
        mesh=plsc.VectorSubcoreMesh(core_axis_name="c", subcore_axis_name="s",
                                    num_cores=NC, num_subcores=NS),
        scratch_types=(
            [pltpu.VMEM((2 * RIDX,), jnp.int32),
             pltpu.VMEM((2 * NB, CHUNK), jnp.int32)]
            + [pltpu.VMEM((CHUNK, D), jnp.float32) for _ in range(NB)]
            + [pltpu.VMEM_SHARED((N_PAD, D), jnp.float32),
               pltpu.SemaphoreType.DMA((2,)),
               pltpu.SemaphoreType.DMA((2,)),
               pltpu.SemaphoreType.DMA((NB,)),
               pltpu.SemaphoreType.DMA((NB,))]),
    )


def _sc_scatter(x, src, dst, zeros):
    return _get_sc_scatter()(x, src, dst, zeros)


def _mlp_body(x_ref, p0_ref, p1_ref, w1_ref, b1_ref, w2_ref, b2_ref, o_ref):
    h = x_ref[...] + p0_ref[0] + p1_ref[0]
    h = jnp.maximum(
        jnp.dot(h, w1_ref[...], preferred_element_type=jnp.float32)
        + b1_ref[...], 0.0)
    h = jnp.dot(h, w2_ref[...], preferred_element_type=jnp.float32) + b2_ref[...]
    o_ref[...] = jnp.maximum(h, 0.0)


def _mlp(xp, P, w1, b1, w2, b2):
    row = pl.BlockSpec((RB, D), lambda i: (i, 0))
    p0s = pl.BlockSpec((1, RB, D), lambda i: (0, i, 0))
    p1s = pl.BlockSpec((1, RB, D), lambda i: (1, i, 0))
    full = pl.BlockSpec((D, D), lambda i: (0, 0))
    vec = pl.BlockSpec((1, D), lambda i: (0, 0))
    return pl.pallas_call(
        _mlp_body,
        grid=(NRB,),
        in_specs=[row, p0s, p1s, full, vec, full, vec],
        out_specs=row,
        out_shape=jax.ShapeDtypeStruct((N, D), jnp.float32),
    )(xp, P, P, w1, b1, w2, b2)


def _final_body(h_ref, q0_ref, q1_ref, b_ref, w1_ref, b1_ref, w2_ref, b2_ref,
                fcw_ref, fcb_ref, hw_ref, hb_ref, o_ref, acc_ref, cnt_ref):
    i = pl.program_id(0)

    @pl.when(i == 0)
    def _():
        acc_ref[...] = jnp.zeros_like(acc_ref)
        cnt_ref[...] = jnp.zeros_like(cnt_ref)

    h = h_ref[...] + q0_ref[0] + q1_ref[0]
    h = jnp.maximum(
        jnp.dot(h, w1_ref[...], preferred_element_type=jnp.float32)
        + b1_ref[...], 0.0)
    h = jnp.dot(h, w2_ref[...], preferred_element_type=jnp.float32) + b2_ref[...]
    h = jnp.maximum(h, 0.0)

    seg = b_ref[0]
    gids = lax.broadcasted_iota(jnp.int32, (G, RB), 0)
    onehot = (gids == seg).astype(jnp.float32)
    acc_ref[...] += jnp.dot(onehot, h, preferred_element_type=jnp.float32)
    cnt_ref[...] += jnp.sum(onehot, axis=1, keepdims=True)

    @pl.when(i == NRB - 1)
    def _():
        counts = jnp.maximum(cnt_ref[...][:, 0:1], 1.0)
        pooled = acc_ref[...] / counts
        s = jnp.maximum(
            jnp.dot(pooled, fcw_ref[...], preferred_element_type=jnp.float32)
            + fcb_ref[...], 0.0)
        o_ref[...] = (jnp.dot(s, hw_ref[...], preferred_element_type=jnp.float32)
                      + hb_ref[...])


def _final(h1, Q, batch3, w1, b1, w2, b2, fcw, fcb, hw, hb):
    row = pl.BlockSpec((RB, D), lambda i: (i, 0))
    q0s = pl.BlockSpec((1, RB, D), lambda i: (0, i, 0))
    q1s = pl.BlockSpec((1, RB, D), lambda i: (1, i, 0))
    full = pl.BlockSpec((D, D), lambda i: (0, 0))
    vec = pl.BlockSpec((1, D), lambda i: (0, 0))
    return pl.pallas_call(
        _final_body,
        grid=(NRB,),
        in_specs=[
            row, q0s, q1s,
            pl.BlockSpec((1, 1, RB), lambda i: (i, 0, 0)),
            full, vec, full, vec,
            full, vec, full, vec,
        ],
        out_specs=pl.BlockSpec((G, D), lambda i: (0, 0)),
        out_shape=jax.ShapeDtypeStruct((G, D), jnp.float32),
        scratch_shapes=[
            pltpu.VMEM((G, D), jnp.float32),
            pltpu.VMEM((G, D), jnp.float32),
        ],
    )(h1, Q, Q, batch3, w1, b1, w2, b2, fcw, fcb, hw, hb)


def kernel(x, edge_index, batch,
           c1_w1, c1_b1, c1_w2, c1_b2,
           c2_w1, c2_b1, c2_w2, c2_b2,
           fc_w, fc_b, hS_w, hS_b, hP_w, hP_b, hN_w, hN_b):
    pad = jnp.arange(E_PAD - E, dtype=jnp.int32)
    src = jnp.concatenate([edge_index[0], pad % N])
    dst = jnp.concatenate([edge_index[1], N + pad % (N_PAD - N)])
    zeros = jnp.zeros((ZROWS, D), jnp.float32)

    P = _sc_scatter(x, src, dst, zeros)
    h1 = _mlp(x, P,
              c1_w1, c1_b1.reshape(1, D), c1_w2, c1_b2.reshape(1, D))
    Q = _sc_scatter(h1, src, dst, zeros)

    batch3 = batch.reshape(NRB, 1, RB)
    fcw = jnp.pad(fc_w, ((0, 0), (0, D - fc_w.shape[1])))
    fcb = jnp.pad(fc_b, (0, D - fc_b.shape[0])).reshape(1, D)
    hw = jnp.concatenate([hS_w, hP_w, hN_w], axis=1)
    hw = jnp.pad(hw, ((0, D - hw.shape[0]), (0, D - hw.shape[1])))
    hb = jnp.concatenate([hS_b, hP_b, hN_b], axis=0)
    hb = jnp.pad(hb, (0, D - hb.shape[0])).reshape(1, D)

    o = _final(h1, Q, batch3,
               c2_w1, c2_b1.reshape(1, D), c2_w2, c2_b2.reshape(1, D),
               fcw, fcb, hw, hb)
    return o[:, 0], o[:, 1], o[:, 2]
